# baseline (speedup 1.0000x reference)
import functools

import jax
import jax.numpy as jnp
from jax import lax
from jax.experimental import pallas as pl
from jax.experimental.pallas import tpu as pltpu
from jax.experimental.pallas import tpu_sc as plsc

_VOCAB = 49508
_EMBED = 768
_SEQ = 77
_SEQP = 80
_BATCH = 4096
_NC = 2
_NS = 16
_NW = _NC * _NS
_ROWS = _BATCH * _SEQP
_RPW = _ROWS // _NW
_CH = 32
_NCHUNK = _RPW // _CH
_NBUF = 4


_mesh = plsc.VectorSubcoreMesh(core_axis_name="c", subcore_axis_name="s")


@functools.partial(
    pl.kernel,
    mesh=_mesh,
    out_type=jax.ShapeDtypeStruct((_ROWS, _EMBED), jnp.float32),
    scratch_types=[
        pltpu.VMEM((_RPW // 128, 128), jnp.int32),
        pltpu.VMEM((_CH, _EMBED), jnp.float32),
        pltpu.VMEM((_CH, _EMBED), jnp.float32),
        pltpu.VMEM((_CH, _EMBED), jnp.float32),
        pltpu.VMEM((_CH, _EMBED), jnp.float32),
        pltpu.SemaphoreType.DMA,
        pltpu.SemaphoreType.DMA,
        pltpu.SemaphoreType.DMA,
        pltpu.SemaphoreType.DMA,
        pltpu.SemaphoreType.DMA,
        pltpu.SemaphoreType.DMA,
        pltpu.SemaphoreType.DMA,
        pltpu.SemaphoreType.DMA,
    ],
)
def _gather_kernel(ids_hbm, tok_hbm, out_hbm, idx_v,
                   b0, b1, b2, b3, i0, i1, i2, i3, o0, o1, o2, o3):
    bufs = (b0, b1, b2, b3)
    isems = (i0, i1, i2, i3)
    osems = (o0, o1, o2, o3)
    wid = lax.axis_index("s") * _NC + lax.axis_index("c")
    rbase = wid * _RPW
    pltpu.sync_copy(ids_hbm.at[wid], idx_v)

    def start_gather(c, buf, sem):
        grow = c // (128 // _CH)
        goff = (c % (128 // _CH)) * _CH
        return pltpu.async_copy(
            tok_hbm.at[idx_v.at[grow, pl.ds(goff, _CH)]], buf, sem)

    for c in range(_NBUF - 1):
        start_gather(c, bufs[c], isems[c])

    def group(t, inner):
        for b in range(_NBUF):
            c = t * _NBUF + b
            pltpu.make_async_copy(tok_hbm.at[idx_v.at[0, pl.ds(0, _CH)]],
                                  bufs[b], isems[b]).wait()
            pltpu.async_copy(bufs[b],
                             out_hbm.at[pl.ds(rbase + c * _CH, _CH)],
                             osems[b])
            cg = c + _NBUF - 1
            bg = (b + _NBUF - 1) % _NBUF

            @pl.when(cg >= _NBUF)
            def _wait_prev():
                pltpu.make_async_copy(bufs[bg],
                                      out_hbm.at[pl.ds(0, _CH)],
                                      osems[bg]).wait()

            @pl.when(cg < _NCHUNK)
            def _issue():
                start_gather(cg, bufs[bg], isems[bg])
        return inner

    lax.fori_loop(0, _NCHUNK // _NBUF, group, 0)
    blast = (_NCHUNK - 1) % _NBUF
    pltpu.make_async_copy(bufs[blast], out_hbm.at[pl.ds(0, _CH)],
                          osems[blast]).wait()


def _addpos_body(x_ref, p_ref, o_ref):
    o_ref[0] = x_ref[...] + p_ref[...]


_addpos_tc = pl.pallas_call(
    _addpos_body,
    grid=(_BATCH,),
    in_specs=[
        pl.BlockSpec((_SEQP, _EMBED), lambda b: (b, 0)),
        pl.BlockSpec((_SEQP, _EMBED), lambda b: (0, 0)),
    ],
    out_specs=pl.BlockSpec((1, _SEQP, _EMBED), lambda b: (b, 0, 0)),
    out_shape=jax.ShapeDtypeStruct((_BATCH, _SEQP, _EMBED), jnp.float32),
)


def kernel(input_ids, token_embedding, position_embedding):
    idsp = jnp.pad(input_ids.astype(jnp.int32),
                   ((0, 0), (0, _SEQP - _SEQ)))
    ids = idsp.reshape(_NW, _RPW // 128, 128)
    gathered = _gather_kernel(ids, token_embedding)
    posp = jnp.pad(position_embedding, ((0, _SEQP - _SEQ), (0, 0)))
    out = _addpos_tc(gathered, posp)
    return out[:, :_SEQ, :]


# speedup vs baseline: 2.1634x; 2.1634x over previous
import functools

import jax
import jax.numpy as jnp
from jax import lax
from jax.experimental import pallas as pl
from jax.experimental.pallas import tpu as pltpu
from jax.experimental.pallas import tpu_sc as plsc

_VOCAB = 49508
_EMBED = 768
_SEQ = 77
_SEQP = 80
_BATCH = 4096
_NC = 2
_NS = 16
_NW = _NC * _NS
_SPW = _BATCH // _NW
_PC = 16
_PH = _SEQP // _PC
_CH = 2 * _PC
_NG = _SPW // 2
_LANES = 16
_PW = _PC * _EMBED
_NBUF = 4


_mesh = plsc.VectorSubcoreMesh(core_axis_name="c", subcore_axis_name="s")


@functools.partial(
    pl.kernel,
    mesh=_mesh,
    out_type=jax.ShapeDtypeStruct((_BATCH, _PH, _PC, _EMBED), jnp.float32),
    scratch_types=[
        pltpu.VMEM((_NG * _CH // 128, 128), jnp.int32),
        pltpu.VMEM((2 * _PW,), jnp.float32),
        pltpu.VMEM((_CH, _EMBED), jnp.float32),
        pltpu.VMEM((_CH, _EMBED), jnp.float32),
        pltpu.VMEM((_CH, _EMBED), jnp.float32),
        pltpu.VMEM((_CH, _EMBED), jnp.float32),
        pltpu.SemaphoreType.DMA,
        pltpu.SemaphoreType.DMA,
        pltpu.SemaphoreType.DMA,
        pltpu.SemaphoreType.DMA,
        pltpu.SemaphoreType.DMA,
        pltpu.SemaphoreType.DMA,
        pltpu.SemaphoreType.DMA,
        pltpu.SemaphoreType.DMA,
    ],
)
def _emb_kernel(ids_hbm, tok_hbm, pos_hbm, out_hbm, idx_v, pos_v,
                b0, b1, b2, b3, i0, i1, i2, i3, o0, o1, o2, o3):
    bufs = (b0, b1, b2, b3)
    isems = (i0, i1, i2, i3)
    osems = (o0, o1, o2, o3)
    wid = lax.axis_index("s") * _NC + lax.axis_index("c")
    sbase = wid * _SPW

    def start_gather(g, buf, sem):
        grow = g // (128 // _CH)
        goff = (g % (128 // _CH)) * _CH
        return pltpu.async_copy(
            tok_hbm.at[idx_v.at[grow, pl.ds(goff, _CH)]], buf, sem)

    def add_pos(buf):
        @plsc.parallel_loop(0, _CH, unroll=4)
        def rbody(r):
            rowb = r * _EMBED
            for k in range(_EMBED // _LANES):
                pv = pos_v[pl.ds(rowb + k * _LANES, _LANES)]
                plsc.addupdate(buf.at[r, pl.ds(k * _LANES, _LANES)], pv)

    def phase(p, carry):
        pltpu.sync_copy(ids_hbm.at[wid, p], idx_v)
        pltpu.sync_copy(pos_hbm.at[pl.ds(p * _PW, _PW)],
                        pos_v.at[pl.ds(0, _PW)])
        pltpu.sync_copy(pos_hbm.at[pl.ds(p * _PW, _PW)],
                        pos_v.at[pl.ds(_PW, _PW)])
        for g in range(_NBUF - 1):
            start_gather(g, bufs[g], isems[g])

        def group(t, inner):
            for b in range(_NBUF):
                g = t * _NBUF + b
                pltpu.make_async_copy(
                    tok_hbm.at[idx_v.at[0, pl.ds(0, _CH)]],
                    bufs[b], isems[b]).wait()
                add_pos(bufs[b])
                row = sbase + 2 * g
                pltpu.async_copy(bufs[b].at[pl.ds(0, _PC)],
                                 out_hbm.at[row, p], osems[b])
                pltpu.async_copy(bufs[b].at[pl.ds(_PC, _PC)],
                                 out_hbm.at[row + 1, p], osems[b])
                cg = g + _NBUF - 1
                bg = (b + _NBUF - 1) % _NBUF

                @pl.when(cg >= _NBUF)
                def _wait_prev():
                    pltpu.make_async_copy(bufs[bg].at[pl.ds(0, _PC)],
                                          out_hbm.at[0, 0],
                                          osems[bg]).wait()
                    pltpu.make_async_copy(bufs[bg].at[pl.ds(0, _PC)],
                                          out_hbm.at[0, 0],
                                          osems[bg]).wait()

                @pl.when(cg < _NG)
                def _issue():
                    start_gather(cg, bufs[bg], isems[bg])
            return inner

        lax.fori_loop(0, _NG // _NBUF, group, 0)
        blast = (_NG - 1) % _NBUF
        pltpu.make_async_copy(bufs[blast].at[pl.ds(0, _PC)],
                              out_hbm.at[0, 0],
                              osems[blast]).wait()
        pltpu.make_async_copy(bufs[blast].at[pl.ds(0, _PC)],
                              out_hbm.at[0, 0],
                              osems[blast]).wait()
        return carry

    lax.fori_loop(0, _PH, phase, 0)


def kernel(input_ids, token_embedding, position_embedding):
    idsp = jnp.pad(input_ids.astype(jnp.int32),
                   ((0, 0), (0, _SEQP - _SEQ)))
    ids = (idsp.reshape(_NW, _NG, 2, _PH, _PC)
           .transpose(0, 3, 1, 2, 4)
           .reshape(_NW, _PH, _NG * _CH // 128, 128))
    pos = jnp.pad(position_embedding,
                  ((0, _SEQP - _SEQ), (0, 0))).reshape(_SEQP * _EMBED)
    out = _emb_kernel(ids, token_embedding, pos)
    return out.reshape(_BATCH, _SEQP, _EMBED)[:, :_SEQ, :]
